# Initial kernel scaffold; baseline (speedup 1.0000x reference)
import jax
import jax.numpy as jnp
from jax.experimental import pallas as pl


def kernel(x, edge_index, batch, W1_1, W2_1, W1_2, W2_2, W1_3, W2_3, C0_W, C0_b, C2_W, C2_b):
    raise NotImplementedError("write your pallas kernel here")


# speedup vs baseline: 8.5288x; 8.5288x over previous
import functools

import jax
import jax.numpy as jnp
from jax import lax
from jax.experimental import pallas as pl
from jax.experimental.pallas import tpu as pltpu
from jax.experimental.pallas import tpu_sc as plsc

_NC = 2
_NS = 16
_NW = _NC * _NS


@functools.lru_cache(maxsize=None)
def _make_sc_gather_add(n_rows, n_out, n_idx, feat, chunk):
    assert n_idx % (_NW * chunk) == 0
    n_chunks = n_idx // (_NW * chunk)
    assert chunk <= 128 and chunk % 8 == 0
    assert feat % 16 == 0
    assert n_out % (_NS * 8) == 0
    rpt = n_out // _NS
    zr = 128 if rpt % 128 == 0 else rpt
    assert rpt % zr == 0

    mesh = plsc.VectorSubcoreMesh(core_axis_name="c", subcore_axis_name="s",
                                  num_cores=_NC, num_subcores=_NS)

    @functools.partial(
        pl.kernel,
        out_type=jax.ShapeDtypeStruct((_NC, n_out, feat), jnp.float32),
        mesh=mesh,
        scratch_types=[
            pltpu.VMEM((n_chunks, chunk), jnp.int32),
            pltpu.VMEM((n_chunks, chunk), jnp.int32),
            pltpu.VMEM((2, chunk, feat), jnp.float32),
            pltpu.VMEM((zr, feat), jnp.float32),
            pltpu.VMEM_SHARED((n_out, feat), jnp.float32),
            pltpu.SemaphoreType.DMA,
            pltpu.SemaphoreType.DMA,
        ],
        compiler_params=pltpu.CompilerParams(use_tc_tiling_on_sc=False),
    )
    def k(table_hbm, src_hbm, dst_hbm, out_hbm,
          src_v, dst_v, rows_v, zer_v, agg_sh, sem0, sem1):
        cid = lax.axis_index("c")
        sid = lax.axis_index("s")
        wid = sid * _NC + cid

        pltpu.sync_copy(src_hbm.at[wid], src_v)
        pltpu.sync_copy(dst_hbm.at[wid], dst_v)

        def zfill(r, _):
            for c in range(feat // 16):
                zer_v[r, pl.ds(c * 16, 16)] = jnp.zeros((16,), jnp.float32)
            return 0
        lax.fori_loop(0, zr, zfill, 0)

        for j in range(rpt // zr):
            pltpu.sync_copy(zer_v, agg_sh.at[pl.ds(sid * rpt + j * zr, zr)])
        plsc.subcore_barrier()

        sems = (sem0, sem1)

        def gather(j, buf):
            pltpu.async_copy(
                table_hbm.at[src_v.at[j]], rows_v.at[buf], sems[buf])

        def wait(j, buf):
            pltpu.make_async_copy(
                table_hbm.at[src_v.at[j]], rows_v.at[buf], sems[buf]).wait()

        def scat(j, buf):
            pltpu.sync_copy(rows_v.at[buf], agg_sh.at[dst_v.at[j]], add=True)

        gather(0, 0)

        def body(jj, _):
            j = jj * 2
            gather(j + 1, 1)
            wait(j, 0)
            scat(j, 0)
            gather(lax.min(j + 2, n_chunks - 1), 0)
            wait(j + 1, 1)
            scat(j + 1, 1)
            return 0

        lax.fori_loop(0, n_chunks // 2, body, 0)
        if n_chunks % 2 == 1:
            wait(n_chunks - 1, 0)
            scat(n_chunks - 1, 0)
        else:
            wait(n_chunks - 1, 0)
        plsc.subcore_barrier()

        pltpu.sync_copy(agg_sh.at[pl.ds(sid * rpt, rpt)],
                        out_hbm.at[cid, pl.ds(sid * rpt, rpt)])

    return k


def _tc_layer(f, w1, aggs, w2s, bn=1000):
    n, din = f.shape
    h = w1.shape[0]
    assert n % bn == 0
    nk = len(aggs)

    def body(*refs):
        f_ref, w1_ref, o_ref = refs[0], refs[1], refs[-1]
        dn = (((1,), (1,)), ((), ()))
        acc = lax.dot_general(f_ref[...], w1_ref[...], dn,
                              preferred_element_type=jnp.float32)
        for i in range(nk):
            a0, a1, w2 = refs[2 + 3 * i], refs[3 + 3 * i], refs[4 + 3 * i]
            acc = acc + lax.dot_general(a0[...] + a1[...], w2[...], dn,
                                        preferred_element_type=jnp.float32)
        o_ref[...] = jnp.maximum(acc, 0.0)

    in_specs = [pl.BlockSpec((bn, din), lambda i: (i, 0)),
                pl.BlockSpec((h, din), lambda i: (0, 0))]
    ops = [f, w1]
    for a, w2 in zip(aggs, w2s):
        dk = w2.shape[1]
        in_specs += [
            pl.BlockSpec((bn, dk), lambda i: (i, 0)),
            pl.BlockSpec((bn, dk), lambda i: (i, 0)),
            pl.BlockSpec((h, dk), lambda i: (0, 0)),
        ]
        ops += [a[0], a[1], w2]

    return pl.pallas_call(
        body,
        grid=(n // bn,),
        in_specs=in_specs,
        out_specs=pl.BlockSpec((bn, h), lambda i: (i, 0)),
        out_shape=jax.ShapeDtypeStruct((n, h), jnp.float32),
    )(*ops)


def _tc_head(p0, p1, c0_w, c0_b, c2_w, c2_b, g, h):
    def body(p0_ref, p1_ref, w0_ref, b0_ref, w2_ref, b2_ref, o_ref):
        p = p0_ref[...] + p1_ref[...]
        sums = p[:g, :h]
        cnt = p[:g, h:h + 1]
        pooled = sums / jnp.maximum(cnt, 1.0)
        dn = (((1,), (1,)), ((), ()))
        z = jnp.maximum(
            lax.dot_general(pooled, w0_ref[...], dn,
                            preferred_element_type=jnp.float32) + b0_ref[...],
            0.0)
        o_ref[...] = lax.dot_general(z, w2_ref[...], dn,
                                     preferred_element_type=jnp.float32) + b2_ref[...]

    w2_pad = jnp.zeros((128, h), jnp.float32).at[0].set(c2_w[0])
    b2_pad = jnp.zeros((g, 128), jnp.float32).at[:, 0].set(c2_b[0])
    out = pl.pallas_call(
        body,
        out_shape=jax.ShapeDtypeStruct((g, 128), jnp.float32),
    )(p0, p1, c0_w,
      jnp.broadcast_to(c0_b.reshape(1, h), (g, h)),
      w2_pad, b2_pad)
    return out[:, :1]


def kernel(x, edge_index, batch, W1_1, W2_1, W1_2, W2_2, W1_3, W2_3,
           C0_W, C0_b, C2_W, C2_b):
    n, d = x.shape
    e = edge_index.shape[1]
    h = W1_1.shape[0]
    g = 64
    chunk = 80
    n_chunks = e // (_NW * chunk)

    src = edge_index[0].reshape(_NW, n_chunks, chunk)
    dst = edge_index[1].reshape(_NW, n_chunks, chunk)

    n_acc = ((n + 127) // 128) * 128
    sc = _make_sc_gather_add(n, n_acc, e, h, chunk)
    assert d == 2 * h
    agg1a = sc(x[:, :h], src, dst)
    agg1b = sc(x[:, h:], src, dst)
    h1 = _tc_layer(x, W1_1, [agg1a, agg1b], [W2_1[:, :h], W2_1[:, h:]])
    h2 = _tc_layer(h1, W1_2, [sc(h1, src, dst)], [W2_2])
    h3 = _tc_layer(h2, W1_3, [sc(h2, src, dst)], [W2_3])

    feat_p = h + 16
    np_ = ((n + _NW * chunk - 1) // (_NW * chunk)) * (_NW * chunk)
    h3a = jnp.zeros((np_, feat_p), jnp.float32)
    h3a = h3a.at[:n, :h].set(h3)
    h3a = h3a.at[:n, h].set(1.0)
    batch_p = jnp.pad(batch, (0, np_ - n), constant_values=g)
    pc = np_ // (_NW * chunk)
    pool_src = jnp.arange(np_, dtype=jnp.int32).reshape(_NW, pc, chunk)
    pool_dst = batch_p.reshape(_NW, pc, chunk)
    n_out_p = 128
    pooled = _make_sc_gather_add(np_, n_out_p, np_, feat_p, chunk)(
        h3a, pool_src, pool_dst)

    return _tc_head(pooled[0], pooled[1], C0_W, C0_b, C2_W, C2_b, g, h)


# speedup vs baseline: 11.0063x; 1.2905x over previous
import functools

import jax
import jax.numpy as jnp
from jax import lax
from jax.experimental import pallas as pl
from jax.experimental.pallas import tpu as pltpu
from jax.experimental.pallas import tpu_sc as plsc

_NC = 2
_NS = 16
_NW = _NC * _NS


@functools.lru_cache(maxsize=None)
def _make_sc_gather_add(n_rows, n_out, n_idx, feat, chunk):
    assert n_idx % (_NW * chunk) == 0
    n_chunks = n_idx // (_NW * chunk)
    assert chunk <= 128 and chunk % 8 == 0
    assert feat % 16 == 0
    assert n_out % (_NS * 8) == 0
    rpt = n_out // _NS
    zr = 128 if rpt % 128 == 0 else rpt
    assert rpt % zr == 0

    mesh = plsc.VectorSubcoreMesh(core_axis_name="c", subcore_axis_name="s",
                                  num_cores=_NC, num_subcores=_NS)

    @functools.partial(
        pl.kernel,
        out_type=jax.ShapeDtypeStruct((_NC, n_out, feat), jnp.float32),
        mesh=mesh,
        scratch_types=[
            pltpu.VMEM((n_chunks, chunk), jnp.int32),
            pltpu.VMEM((n_chunks, chunk), jnp.int32),
            pltpu.VMEM((2, chunk, feat), jnp.float32),
            pltpu.VMEM((zr, feat), jnp.float32),
            pltpu.VMEM_SHARED((n_out, feat), jnp.float32),
            pltpu.SemaphoreType.DMA,
            pltpu.SemaphoreType.DMA,
        ],
        compiler_params=pltpu.CompilerParams(use_tc_tiling_on_sc=False),
    )
    def k(table_hbm, src_hbm, dst_hbm, out_hbm,
          src_v, dst_v, rows_v, zer_v, agg_sh, sem0, sem1):
        cid = lax.axis_index("c")
        sid = lax.axis_index("s")
        wid = sid * _NC + cid

        pltpu.sync_copy(src_hbm.at[wid], src_v)
        pltpu.sync_copy(dst_hbm.at[wid], dst_v)

        sems = (sem0, sem1)

        def gather(j, buf):
            pltpu.async_copy(
                table_hbm.at[src_v.at[j]], rows_v.at[buf], sems[buf])

        def wait(j, buf):
            pltpu.make_async_copy(
                table_hbm.at[src_v.at[j]], rows_v.at[buf], sems[buf]).wait()

        def scat(j, buf):
            pltpu.sync_copy(rows_v.at[buf], agg_sh.at[dst_v.at[j]], add=True)

        gather(0, 0)

        def zfill(r, _):
            for c in range(feat // 16):
                zer_v[r, pl.ds(c * 16, 16)] = jnp.zeros((16,), jnp.float32)
            return 0
        lax.fori_loop(0, zr, zfill, 0)

        for j in range(rpt // zr):
            pltpu.sync_copy(zer_v, agg_sh.at[pl.ds(sid * rpt + j * zr, zr)])
        plsc.subcore_barrier()

        def body(jj, _):
            j = jj * 2
            gather(j + 1, 1)
            wait(j, 0)
            scat(j, 0)
            gather(lax.min(j + 2, n_chunks - 1), 0)
            wait(j + 1, 1)
            scat(j + 1, 1)
            return 0

        lax.fori_loop(0, n_chunks // 2, body, 0)
        if n_chunks % 2 == 1:
            wait(n_chunks - 1, 0)
            scat(n_chunks - 1, 0)
        else:
            wait(n_chunks - 1, 0)
        plsc.subcore_barrier()

        pltpu.sync_copy(agg_sh.at[pl.ds(sid * rpt, rpt)],
                        out_hbm.at[cid, pl.ds(sid * rpt, rpt)])

    return k


_DN = (((1,), (1,)), ((), ()))


def _tc_fy(wc, f, parts, bn=1024):
    n, din = f.shape
    h2 = wc.shape[0]
    h = h2 // 2

    def body(*refs):
        f_ref, o1_ref, o2_ref = refs[0], refs[-2], refs[-1]
        v = f_ref[...]
        if parts is not None:
            v = jnp.maximum(v + refs[1][...] + refs[2][...], 0.0)
        w_ref = refs[-3]
        acc = lax.dot_general(v, w_ref[...], _DN,
                              preferred_element_type=jnp.float32,
                              precision=lax.Precision.HIGHEST)
        o1_ref[...] = acc[:, :h]
        o2_ref[...] = acc[:, h:]

    in_specs = [pl.BlockSpec((bn, din), lambda i: (i, 0))]
    ops = [f]
    if parts is not None:
        in_specs += [pl.BlockSpec((bn, din), lambda i: (i, 0))] * 2
        ops += [parts[0], parts[1]]
    in_specs.append(pl.BlockSpec((h2, din), lambda i: (0, 0)))
    ops.append(wc)

    return pl.pallas_call(
        body,
        grid=(n // bn,),
        in_specs=in_specs,
        out_specs=[pl.BlockSpec((bn, h), lambda i: (i, 0))] * 2,
        out_shape=[jax.ShapeDtypeStruct((n, h), jnp.float32)] * 2,
    )(*ops)


def _tc_poolin(f, parts, feat_p, bn=1024):
    n, h = f.shape

    def body(f_ref, p0_ref, p1_ref, o_ref):
        v = jnp.maximum(f_ref[...] + p0_ref[...] + p1_ref[...], 0.0)
        o_ref[...] = jnp.concatenate(
            [v, jnp.ones((bn, 1), jnp.float32),
             jnp.zeros((bn, feat_p - h - 1), jnp.float32)], axis=1)

    return pl.pallas_call(
        body,
        grid=(n // bn,),
        in_specs=[pl.BlockSpec((bn, h), lambda i: (i, 0))] * 3,
        out_specs=pl.BlockSpec((bn, feat_p), lambda i: (i, 0)),
        out_shape=jax.ShapeDtypeStruct((n, feat_p), jnp.float32),
    )(f, parts[0], parts[1])


def _tc_head(p0, p1, c0_w, c0_b, c2_w, c2_b, g, h):
    def body(p0_ref, p1_ref, w0_ref, b0_ref, w2_ref, b2_ref, o_ref):
        p = p0_ref[...] + p1_ref[...]
        sums = p[:g, :h]
        cnt = p[:g, h:h + 1]
        pooled = sums / jnp.maximum(cnt, 1.0)
        z = jnp.maximum(
            lax.dot_general(pooled, w0_ref[...], _DN,
                            preferred_element_type=jnp.float32,
                            precision=lax.Precision.HIGHEST) + b0_ref[...],
            0.0)
        o_ref[...] = lax.dot_general(z, w2_ref[...], _DN,
                                     preferred_element_type=jnp.float32,
                                     precision=lax.Precision.HIGHEST) + b2_ref[...]

    w2_pad = jnp.zeros((128, h), jnp.float32).at[0].set(c2_w[0])
    b2_pad = jnp.zeros((g, 128), jnp.float32).at[:, 0].set(c2_b[0])
    out = pl.pallas_call(
        body,
        out_shape=jax.ShapeDtypeStruct((g, 128), jnp.float32),
    )(p0, p1, c0_w,
      jnp.broadcast_to(c0_b.reshape(1, h), (g, h)),
      w2_pad, b2_pad)
    return out[:, :1]


def kernel(x, edge_index, batch, W1_1, W2_1, W1_2, W2_2, W1_3, W2_3,
           C0_W, C0_b, C2_W, C2_b):
    n, d = x.shape
    e = edge_index.shape[1]
    h = W1_1.shape[0]
    g = 64
    chunk = 80
    np_ = ((n + _NW * chunk - 1) // (_NW * chunk)) * (_NW * chunk)

    ep = ((e + _NW * chunk - 1) // (_NW * chunk)) * (_NW * chunk)
    n_chunks = ep // (_NW * chunk)
    src = jnp.pad(edge_index[0], (0, ep - e)).reshape(_NW, n_chunks, chunk)
    dst = jnp.pad(edge_index[1], (0, ep - e),
                  constant_values=np_ - 1).reshape(_NW, n_chunks, chunk)
    sc = _make_sc_gather_add(np_, np_, ep, h, chunk)

    x_p = jnp.pad(x, ((0, np_ - n), (0, 0)))
    wc1 = jnp.concatenate([W1_1, W2_1], axis=0)
    wc2 = jnp.concatenate([W1_2, W2_2], axis=0)
    wc3 = jnp.concatenate([W1_3, W2_3], axis=0)

    f1, y1 = _tc_fy(wc1, x_p, None)
    p1 = sc(y1, src, dst)
    f2, y2 = _tc_fy(wc2, f1, p1)
    p2 = sc(y2, src, dst)
    f3, y3 = _tc_fy(wc3, f2, p2)
    p3 = sc(y3, src, dst)

    feat_p = h + 16
    h3a = _tc_poolin(f3, p3, feat_p)
    batch_p = jnp.pad(batch, (0, np_ - n), constant_values=g)
    pc = np_ // (_NW * chunk)
    pool_src = jnp.arange(np_, dtype=jnp.int32).reshape(_NW, pc, chunk)
    pool_dst = batch_p.reshape(_NW, pc, chunk)
    n_out_p = 128
    pooled = _make_sc_gather_add(np_, n_out_p, np_, feat_p, chunk)(
        h3a, pool_src, pool_dst)

    return _tc_head(pooled[0], pooled[1], C0_W, C0_b, C2_W, C2_b, g, h)
